# Initial kernel scaffold; baseline (speedup 1.0000x reference)
import jax
import jax.numpy as jnp
from jax.experimental import pallas as pl


def kernel(x, position_ids, cos_cached, sin_cached):
    raise NotImplementedError("write your pallas kernel here")


# speedup vs baseline: 1.4722x; 1.4722x over previous
import functools

import jax
import jax.numpy as jnp
from jax import lax
from jax.experimental import pallas as pl
from jax.experimental.pallas import tpu as pltpu
from jax.experimental.pallas import tpu_sc as plsc

NC = 2
NS = 16
NW = NC * NS

B = 16384
D = 128
CHUNK = 128
B_PER_W = B // NW
NCHUNK = B_PER_W // CHUNK


def _gather_body(pos_hbm, cos_hbm, sin_hbm, cos_out, sin_out,
                 idx_v, cos_buf, sin_buf, sem_c, sem_s):
    wid = lax.axis_index("s") * NC + lax.axis_index("c")
    pltpu.sync_copy(pos_hbm.at[pl.ds(wid * NCHUNK, NCHUNK)], idx_v)

    for c in range(NCHUNK):
        cg = pltpu.async_copy(cos_hbm.at[idx_v.at[c]], cos_buf.at[c % 2], sem_c)
        sg = pltpu.async_copy(sin_hbm.at[idx_v.at[c]], sin_buf.at[c % 2], sem_s)
        cg.wait()
        sg.wait()
        base = wid * B_PER_W + c * CHUNK
        pltpu.sync_copy(cos_buf.at[c % 2], cos_out.at[pl.ds(base, CHUNK)])
        pltpu.sync_copy(sin_buf.at[c % 2], sin_out.at[pl.ds(base, CHUNK)])


@functools.partial(jax.jit, static_argnames=())
def _rope_gather(pos, cos_cached, sin_cached):
    mesh = plsc.VectorSubcoreMesh(core_axis_name="c", subcore_axis_name="s")
    out_type = (
        jax.ShapeDtypeStruct((B, D), jnp.float32),
        jax.ShapeDtypeStruct((B, D), jnp.float32),
    )
    scratch = [
        pltpu.VMEM((NCHUNK, CHUNK), jnp.int32),
        pltpu.VMEM((2, CHUNK, D), jnp.float32),
        pltpu.VMEM((2, CHUNK, D), jnp.float32),
        pltpu.SemaphoreType.DMA,
        pltpu.SemaphoreType.DMA,
    ]
    return pl.kernel(
        _gather_body,
        out_type=out_type,
        mesh=mesh,
        scratch_types=scratch,
    )(pos, cos_cached, sin_cached)


def kernel(x, position_ids, cos_cached, sin_cached):
    pos = position_ids.reshape(B // CHUNK, CHUNK)
    cos_flat, sin_flat = _rope_gather(pos, cos_cached, sin_cached)
    shape = position_ids.shape + (D,)
    return (cos_flat.reshape(shape).astype(x.dtype),
            sin_flat.reshape(shape).astype(x.dtype))


# speedup vs baseline: 1.5996x; 1.0865x over previous
import functools

import jax
import jax.numpy as jnp
from jax import lax
from jax.experimental import pallas as pl
from jax.experimental.pallas import tpu as pltpu
from jax.experimental.pallas import tpu_sc as plsc

NC = 2
NS = 16
NW = NC * NS

B = 16384
D = 128
CHUNK = 128
B_PER_W = B // NW
NCHUNK = B_PER_W // CHUNK


NSLOT = 3


def _gather_body(pos_hbm, cos_hbm, sin_hbm, cos_out, sin_out,
                 idx_v, cos_buf, sin_buf, *sems):
    sem_gc = sems[0:NSLOT]
    sem_gs = sems[NSLOT:2 * NSLOT]
    sem_wc = sems[2 * NSLOT:3 * NSLOT]
    sem_ws = sems[3 * NSLOT:4 * NSLOT]

    wid = lax.axis_index("s") * NC + lax.axis_index("c")
    pltpu.sync_copy(pos_hbm.at[pl.ds(wid * NCHUNK, NCHUNK)], idx_v)

    gc, gs, wc, ws = {}, {}, {}, {}

    def issue_gather(c):
        s = c % NSLOT
        gc[c] = pltpu.async_copy(cos_hbm.at[idx_v.at[c]], cos_buf.at[s], sem_gc[s])
        gs[c] = pltpu.async_copy(sin_hbm.at[idx_v.at[c]], sin_buf.at[s], sem_gs[s])

    for c in range(min(NSLOT, NCHUNK)):
        issue_gather(c)
    for c in range(NCHUNK):
        s = c % NSLOT
        gc[c].wait()
        gs[c].wait()
        base = wid * B_PER_W + c * CHUNK
        wc[c] = pltpu.async_copy(cos_buf.at[s], cos_out.at[pl.ds(base, CHUNK)], sem_wc[s])
        ws[c] = pltpu.async_copy(sin_buf.at[s], sin_out.at[pl.ds(base, CHUNK)], sem_ws[s])
        nxt = c + NSLOT
        if nxt < NCHUNK:
            wc[c].wait()
            ws[c].wait()
            issue_gather(nxt)
    for c in range(max(0, NCHUNK - NSLOT), NCHUNK):
        wc[c].wait()
        ws[c].wait()


@functools.partial(jax.jit, static_argnames=())
def _rope_gather(pos, cos_cached, sin_cached):
    mesh = plsc.VectorSubcoreMesh(core_axis_name="c", subcore_axis_name="s")
    out_type = (
        jax.ShapeDtypeStruct((B, D), jnp.float32),
        jax.ShapeDtypeStruct((B, D), jnp.float32),
    )
    scratch = [
        pltpu.VMEM((NCHUNK, CHUNK), jnp.int32),
        pltpu.VMEM((NSLOT, CHUNK, D), jnp.float32),
        pltpu.VMEM((NSLOT, CHUNK, D), jnp.float32),
    ] + [pltpu.SemaphoreType.DMA] * (4 * NSLOT)
    return pl.kernel(
        _gather_body,
        out_type=out_type,
        mesh=mesh,
        scratch_types=scratch,
    )(pos, cos_cached, sin_cached)


def kernel(x, position_ids, cos_cached, sin_cached):
    pos = position_ids.reshape(B // CHUNK, CHUNK)
    cos_flat, sin_flat = _rope_gather(pos, cos_cached, sin_cached)
    shape = position_ids.shape + (D,)
    return (cos_flat.reshape(shape).astype(x.dtype),
            sin_flat.reshape(shape).astype(x.dtype))


# speedup vs baseline: 1.6038x; 1.0026x over previous
import jax
import jax.numpy as jnp
from jax import lax
from jax.experimental import pallas as pl
from jax.experimental.pallas import tpu as pltpu
from jax.experimental.pallas import tpu_sc as plsc

NC = 2
NS = 16
NW = NC * NS

B = 16384
D = 128
CHUNK = 128
B_PER_W = B // NW
NCHUNK = B_PER_W // CHUNK
NSLOT = 3


def _gather_body(pos_hbm, cos_hbm, sin_hbm, cos_out, sin_out,
                 idx_v, cos_buf, sin_buf, *sems):
    sem_gc = sems[0:NSLOT]
    sem_gs = sems[NSLOT:2 * NSLOT]
    sem_wc = sems[2 * NSLOT:3 * NSLOT]
    sem_ws = sems[3 * NSLOT:4 * NSLOT]

    wid = lax.axis_index("s") * NC + lax.axis_index("c")
    pltpu.sync_copy(pos_hbm.at[pl.ds(wid * NCHUNK, NCHUNK)], idx_v)

    gc, gs, wc, ws = {}, {}, {}, {}

    def issue_gather(c):
        s = c % NSLOT
        gc[c] = pltpu.async_copy(cos_hbm.at[idx_v.at[c]], cos_buf.at[s], sem_gc[s])
        gs[c] = pltpu.async_copy(sin_hbm.at[idx_v.at[c]], sin_buf.at[s], sem_gs[s])

    for c in range(min(NSLOT, NCHUNK)):
        issue_gather(c)
    for c in range(NCHUNK):
        s = c % NSLOT
        gc[c].wait()
        gs[c].wait()
        base = wid * B_PER_W + c * CHUNK
        wc[c] = pltpu.async_copy(cos_buf.at[s], cos_out.at[pl.ds(base, CHUNK)], sem_wc[s])
        ws[c] = pltpu.async_copy(sin_buf.at[s], sin_out.at[pl.ds(base, CHUNK)], sem_ws[s])
        nxt = c + NSLOT
        if nxt < NCHUNK:
            wc[c].wait()
            ws[c].wait()
            issue_gather(nxt)
    for c in range(max(0, NCHUNK - NSLOT), NCHUNK):
        wc[c].wait()
        ws[c].wait()


@jax.jit
def _rope_gather(pos, cos_cached, sin_cached):
    mesh = plsc.VectorSubcoreMesh(core_axis_name="c", subcore_axis_name="s")
    out_type = (
        jax.ShapeDtypeStruct((B, D), jnp.float32),
        jax.ShapeDtypeStruct((B, D), jnp.float32),
    )
    scratch = [
        pltpu.VMEM((NCHUNK, CHUNK), jnp.int32),
        pltpu.VMEM((NSLOT, CHUNK, D), jnp.float32),
        pltpu.VMEM((NSLOT, CHUNK, D), jnp.float32),
    ] + [pltpu.SemaphoreType.DMA] * (4 * NSLOT)
    return pl.kernel(
        _gather_body,
        out_type=out_type,
        mesh=mesh,
        scratch_types=scratch,
    )(pos, cos_cached, sin_cached)


def kernel(x, position_ids, cos_cached, sin_cached):
    pos = position_ids.reshape(B // CHUNK, CHUNK)
    cos_flat, sin_flat = _rope_gather(pos, cos_cached, sin_cached)
    shape = position_ids.shape + (D,)
    return (cos_flat.reshape(shape).astype(x.dtype),
            sin_flat.reshape(shape).astype(x.dtype))
